# Initial kernel scaffold; baseline (speedup 1.0000x reference)
import jax
import jax.numpy as jnp
from jax.experimental import pallas as pl


def kernel(x, edge_index, W, b):
    raise NotImplementedError("write your pallas kernel here")


# speedup vs baseline: 3.9459x; 3.9459x over previous
import functools

import jax
import jax.numpy as jnp
from jax import lax
from jax.experimental import pallas as pl
from jax.experimental.pallas import tpu as pltpu
from jax.experimental.pallas import tpu_sc as plsc

N_NODES = 10000
N_EDGES = 160000
D_FEAT = 256
D_OUT = 256
H = 128
NC = 2
NS = 16
CHUNK = 128
CPT = -(-N_EDGES // (NS * CHUNK))
E_PAD = NS * CHUNK * CPT
AGG_ROWS = 10240
ZSTRIPE = AGG_ROWS // NS


def _sc_agg_kernel():
    mesh = plsc.VectorSubcoreMesh(core_axis_name="c", subcore_axis_name="s")

    @functools.partial(
        pl.kernel,
        out_type=jax.ShapeDtypeStruct((NC, AGG_ROWS, H), jnp.float32),
        mesh=mesh,
        scratch_types=[
            pltpu.VMEM((CPT, CHUNK), jnp.int32),
            pltpu.VMEM((CPT, CHUNK), jnp.int32),
            pltpu.VMEM((CHUNK, H), jnp.float32),
            pltpu.VMEM_SHARED((AGG_ROWS, H), jnp.float32),
            pltpu.SemaphoreType.DMA,
        ],
    )
    def sc_agg(x_hbm, col_hbm, row_hbm, z_hbm, out_hbm,
               colv, rowv, buf, aggsh, gsem):
        cid = lax.axis_index("c")
        sid = lax.axis_index("s")
        pltpu.sync_copy(col_hbm.at[cid, sid], colv)
        pltpu.sync_copy(row_hbm.at[sid], rowv)
        pltpu.sync_copy(z_hbm, aggsh.at[pl.ds(sid * ZSTRIPE, ZSTRIPE)])
        plsc.subcore_barrier()

        def body(g, carry):
            pltpu.async_copy(x_hbm.at[colv.at[g]], buf, gsem).wait()
            pltpu.sync_copy(buf, aggsh.at[rowv.at[g]], add=True)
            return carry

        lax.fori_loop(0, CPT, body, 0, unroll=False)
        plsc.subcore_barrier()
        pltpu.sync_copy(aggsh.at[pl.ds(sid * ZSTRIPE, ZSTRIPE)],
                        out_hbm.at[cid, pl.ds(sid * ZSTRIPE, ZSTRIPE)])

    return sc_agg


_SC_AGG = _sc_agg_kernel()

BM = 1000


def _tc_body(x_ref, a0_ref, a1_ref, w1_ref, w2a_ref, w2b_ref, b_ref, o_ref):
    acc = jnp.dot(x_ref[...], w1_ref[...], preferred_element_type=jnp.float32)
    acc += jnp.dot(a0_ref[...], w2a_ref[...], preferred_element_type=jnp.float32)
    acc += jnp.dot(a1_ref[...], w2b_ref[...], preferred_element_type=jnp.float32)
    acc += b_ref[...]
    o_ref[...] = jnp.maximum(acc, 0.0)


def _tc_linear(x, a0, a1, w1, w2a, w2b, b2d):
    grid = (N_NODES // BM,)
    return pl.pallas_call(
        _tc_body,
        grid=grid,
        in_specs=[
            pl.BlockSpec((BM, D_FEAT), lambda i: (i, 0)),
            pl.BlockSpec((BM, H), lambda i: (i, 0)),
            pl.BlockSpec((BM, H), lambda i: (i, 0)),
            pl.BlockSpec((D_FEAT, D_OUT), lambda i: (0, 0)),
            pl.BlockSpec((H, D_OUT), lambda i: (0, 0)),
            pl.BlockSpec((H, D_OUT), lambda i: (0, 0)),
            pl.BlockSpec((1, D_OUT), lambda i: (0, 0)),
        ],
        out_specs=pl.BlockSpec((BM, D_OUT), lambda i: (i, 0)),
        out_shape=jax.ShapeDtypeStruct((N_NODES, D_OUT), jnp.float32),
    )(x, a0, a1, w1, w2a, w2b, b2d)


def kernel(x, edge_index, W, b):
    row = edge_index[0].astype(jnp.int32)
    col = edge_index[1].astype(jnp.int32)
    pad = E_PAD - N_EDGES
    row_p = jnp.concatenate([row, jnp.full((pad,), N_NODES, jnp.int32)])
    row_p = row_p.reshape(NS, CPT, CHUNK)
    colb = jnp.concatenate([col, jnp.zeros((pad,), jnp.int32)])
    col_p = jnp.stack([colb, colb + N_NODES]).reshape(NC, NS, CPT, CHUNK)
    x_flat = jnp.concatenate([x[:, :H], x[:, H:]], axis=0)
    zeros = jnp.zeros((ZSTRIPE, H), jnp.float32)

    agg = _SC_AGG(x_flat, col_p, row_p, zeros)

    w1 = W[:D_FEAT]
    w2a = W[D_FEAT:D_FEAT + H]
    w2b = W[D_FEAT + H:]
    out = _tc_linear(x, agg[0, :N_NODES], agg[1, :N_NODES], w1, w2a, w2b,
                     b.reshape(1, D_OUT))
    return out
